# baseline (speedup 1.0000x reference)
import functools

import jax
import jax.numpy as jnp
from jax import lax
from jax.experimental import pallas as pl
from jax.experimental.pallas import tpu as pltpu
from jax.experimental.pallas import tpu_sc as plsc

BS, N, CIN, COUT = 4, 4096, 64, 64
K = 16
BR = 512
BR2 = 1024
VPAD = 8
INF = float("inf")


def _topk_body(c, with_linear, fr_ref, faT_ref, *rest):
    if with_linear:
        w_ref, b_ref, ws_ref, idx_ref, fc_ref, fs_ref, fste_ref = rest
    else:
        (idx_ref,) = rest
    b = pl.program_id(0)
    i = pl.program_id(1)
    fr = fr_ref[0]
    faT = faT_ref[0]
    inner = lax.dot_general(fr, faT, (((1,), (0,)), ((), ())),
                            preferred_element_type=jnp.float32)
    qa = jnp.sum(faT * faT, axis=0, keepdims=True)
    qr = jnp.sum(fr * fr, axis=1, keepdims=True)
    d = inner * -2.0 + qa + qr
    col = lax.broadcasted_iota(jnp.int32, (BR, N), 1)
    rowg = lax.broadcasted_iota(jnp.int32, (BR, N), 0) + i * BR
    d = jnp.where(col == rowg, INF, d)
    lane16 = lax.broadcasted_iota(jnp.int32, (BR, K), 1)
    acc0 = jnp.zeros((BR, K), jnp.int32)

    G, GW = N // 128, 128
    lane32 = lax.broadcasted_iota(jnp.int32, (BR, G), 1)
    laneg = lax.broadcasted_iota(jnp.int32, (BR, GW), 1)
    gm_l, gc_l = [], []
    for g in range(G):
        sub = d[:, g * GW:(g + 1) * GW]
        m = jnp.min(sub, axis=1, keepdims=True)
        c = jnp.min(jnp.where(sub == m, laneg + g * GW, N), axis=1,
                    keepdims=True)
        gm_l.append(m)
        gc_l.append(c)
    gmin0 = jnp.concatenate(gm_l, axis=1)
    gcol0 = jnp.concatenate(gc_l, axis=1)

    def step(t, carry):
        gmin, gcol, acc = carry
        mv = jnp.min(gmin, axis=1, keepdims=True)
        am = jnp.min(jnp.where(gmin == mv, gcol, N), axis=1, keepdims=True)
        acc = jnp.where(lane16 == t, am, acc)
        gsel = lax.shift_right_logical(am, 7)
        sel_rows = []
        for r in range(BR // 8):
            rs = slice(r * 8, r * 8 + 8)
            level = [d[rs, g * GW:(g + 1) * GW] for g in range(G)]
            bitv = gsel[rs]
            nbit = 0
            while len(level) > 1:
                bit = lax.shift_right_logical(bitv, nbit) & 1
                level = [jnp.where(bit == 1, level[2 * i + 1], level[2 * i])
                         for i in range(len(level) // 2)]
                nbit += 1
            sel_rows.append(level[0])
        sel = jnp.concatenate(sel_rows, axis=0)
        colg = laneg + lax.shift_left(gsel, 7)
        ok = (sel > mv) | ((sel == mv) & (colg > am))
        selm = jnp.where(ok, sel, INF)
        nm = jnp.min(selm, axis=1, keepdims=True)
        nc = jnp.min(jnp.where(selm == nm, colg, N), axis=1, keepdims=True)
        gmin = jnp.where(lane32 == gsel, nm, gmin)
        gcol = jnp.where(lane32 == gsel, nc, gcol)
        return (gmin, gcol, acc)

    _, _, acc = lax.fori_loop(0, K, step, (gmin0, gcol0, acc0))
    idx_ref[0] = acc + b * N

    if with_linear:
        fm2 = lax.dot_general(fr, w_ref[...], (((1,), (0,)), ((), ())),
                              preferred_element_type=jnp.float32) + b_ref[0][None, :]
        fc_ref[0] = fm2[:, :COUT]
        fs_ref[0] = fm2[:, COUT:]
        fste_ref[0] = lax.dot_general(fr, ws_ref[...], (((1,), (0,)), ((), ())),
                                      preferred_element_type=jnp.float32)


def _knn_feat_linear(feature_map, featT, weights, bias2, W_ste):
    grid = (BS, N // BR)
    body = functools.partial(_topk_body, CIN, True)
    return pl.pallas_call(
        body,
        grid=grid,
        compiler_params=pltpu.CompilerParams(
            dimension_semantics=("parallel", "parallel")),
        in_specs=[
            pl.BlockSpec((1, BR, CIN), lambda b, i: (b, i, 0)),
            pl.BlockSpec((1, CIN, N), lambda b, i: (b, 0, 0)),
            pl.BlockSpec((CIN, 2 * COUT), lambda b, i: (0, 0)),
            pl.BlockSpec((1, 2 * COUT), lambda b, i: (0, 0)),
            pl.BlockSpec((CIN, COUT), lambda b, i: (0, 0)),
        ],
        out_specs=[
            pl.BlockSpec((1, BR, K), lambda b, i: (b, i, 0)),
            pl.BlockSpec((1, BR, COUT), lambda b, i: (b, i, 0)),
            pl.BlockSpec((1, BR, COUT), lambda b, i: (b, i, 0)),
            pl.BlockSpec((1, BR, COUT), lambda b, i: (b, i, 0)),
        ],
        out_shape=[
            jax.ShapeDtypeStruct((BS, N, K), jnp.int32),
            jax.ShapeDtypeStruct((BS, N, COUT), jnp.float32),
            jax.ShapeDtypeStruct((BS, N, COUT), jnp.float32),
            jax.ShapeDtypeStruct((BS, N, COUT), jnp.float32),
        ],
    )(feature_map, featT, weights, bias2, W_ste)


def _knn_vert(vpad, vpadT):
    grid = (BS, N // BR)
    body = functools.partial(_topk_body, VPAD, False)
    return pl.pallas_call(
        body,
        grid=grid,
        compiler_params=pltpu.CompilerParams(
            dimension_semantics=("parallel", "parallel")),
        in_specs=[
            pl.BlockSpec((1, BR, VPAD), lambda b, i: (b, i, 0)),
            pl.BlockSpec((1, VPAD, N), lambda b, i: (b, 0, 0)),
        ],
        out_specs=pl.BlockSpec((1, BR, K), lambda b, i: (b, i, 0)),
        out_shape=jax.ShapeDtypeStruct((BS, N, K), jnp.int32),
    )(vpad, vpadT)


def _sc_gather_pair(table_v, table_s, idx):
    B = idx.shape[0]
    info = plsc.get_sparse_core_info()
    NC, NS = info.num_cores, info.num_subcores
    NW = NC * NS
    bpw = B // NW
    CH = 512
    mesh = plsc.VectorSubcoreMesh(core_axis_name="c", subcore_axis_name="s")

    @functools.partial(
        pl.kernel, mesh=mesh,
        compiler_params=pltpu.CompilerParams(use_tc_tiling_on_sc=False),
        out_type=(jax.ShapeDtypeStruct((B, VPAD), jnp.float32),
                  jax.ShapeDtypeStruct((B, COUT), jnp.float32)),
        scratch_types=[
            pltpu.VMEM((CH,), jnp.int32),
            pltpu.VMEM((CH, VPAD), jnp.float32),
            pltpu.VMEM((CH, COUT), jnp.float32),
            pltpu.SemaphoreType.DMA,
        ],
    )
    def k(tv_hbm, ts_hbm, idx_hbm, ov_hbm, os_hbm, idx_v, rv, rs, sem):
        wid = lax.axis_index("s") * NC + lax.axis_index("c")
        base = wid * bpw

        def body(c, carry):
            off = base + c * CH
            pltpu.sync_copy(idx_hbm.at[pl.ds(off, CH)], idx_v)
            pltpu.async_copy(tv_hbm.at[idx_v], rv, sem).wait()
            pltpu.async_copy(ts_hbm.at[idx_v], rs, sem).wait()
            pltpu.sync_copy(rv, ov_hbm.at[pl.ds(off, CH)])
            pltpu.sync_copy(rs, os_hbm.at[pl.ds(off, CH)])
            return carry

        lax.fori_loop(0, bpw // CH, body, 0)

    return k(table_v, table_s, idx)


def _sc_gather_one(table, idx):
    B = idx.shape[0]
    info = plsc.get_sparse_core_info()
    NC, NS = info.num_cores, info.num_subcores
    NW = NC * NS
    bpw = B // NW
    CH = 512
    mesh = plsc.VectorSubcoreMesh(core_axis_name="c", subcore_axis_name="s")

    @functools.partial(
        pl.kernel, mesh=mesh,
        compiler_params=pltpu.CompilerParams(use_tc_tiling_on_sc=False),
        out_type=jax.ShapeDtypeStruct((B, COUT), jnp.float32),
        scratch_types=[
            pltpu.VMEM((CH,), jnp.int32),
            pltpu.VMEM((CH, COUT), jnp.float32),
            pltpu.SemaphoreType.DMA,
        ],
    )
    def k(t_hbm, idx_hbm, o_hbm, idx_v, rows, sem):
        wid = lax.axis_index("s") * NC + lax.axis_index("c")
        base = wid * bpw

        def body(c, carry):
            off = base + c * CH
            pltpu.sync_copy(idx_hbm.at[pl.ds(off, CH)], idx_v)
            pltpu.async_copy(t_hbm.at[idx_v], rows, sem).wait()
            pltpu.sync_copy(rows, o_hbm.at[pl.ds(off, CH)])
            return carry

        lax.fori_loop(0, bpw // CH, body, 0)

    return k(table, idx)


def _combine_body(vg_ref, fsg_ref, vr_ref, fc_ref, dir_ref, out_ref):
    v = vr_ref[0]
    dirs = dir_ref[...]
    dn = jnp.sqrt(jnp.sum(dirs * dirs, axis=0, keepdims=True))
    sd = dirs / jnp.maximum(dn, 1e-12)
    acc = None
    for t in range(K):
        dv = vg_ref[0, t] - v
        nrm = jnp.sqrt(jnp.sum(dv * dv, axis=1, keepdims=True))
        rf = dv / jnp.maximum(nrm, 1e-12)
        theta = jnp.maximum(
            lax.dot_general(rf, sd, (((1,), (0,)), ((), ())),
                            preferred_element_type=jnp.float32), 0.0)
        a = theta * fsg_ref[0, t]
        acc = a if acc is None else jnp.maximum(acc, a)
    out_ref[0] = fc_ref[0] + acc


def _combine(vg4, fsg4, vpad, f_center, dirpad):
    grid = (BS, N // BR2)
    return pl.pallas_call(
        _combine_body,
        grid=grid,
        compiler_params=pltpu.CompilerParams(
            dimension_semantics=("parallel", "parallel")),
        in_specs=[
            pl.BlockSpec((1, K, BR2, VPAD), lambda b, i: (b, 0, i, 0)),
            pl.BlockSpec((1, K, BR2, COUT), lambda b, i: (b, 0, i, 0)),
            pl.BlockSpec((1, BR2, VPAD), lambda b, i: (b, i, 0)),
            pl.BlockSpec((1, BR2, COUT), lambda b, i: (b, i, 0)),
            pl.BlockSpec((VPAD, COUT), lambda b, i: (0, 0)),
        ],
        out_specs=pl.BlockSpec((1, BR2, COUT), lambda b, i: (b, i, 0)),
        out_shape=jax.ShapeDtypeStruct((BS, N, COUT), jnp.float32),
    )(vg4, fsg4, vpad, f_center, dirpad)


BRF = 1024


def _pool_body(fg2_ref, sum_ref):
    i = pl.program_id(1)
    fg = fg2_ref[0, 0]
    for t in range(1, K):
        fg = jnp.maximum(fg, fg2_ref[0, t])
    psum = jnp.sum(fg, axis=0, keepdims=True)

    @pl.when(i == 0)
    def _():
        sum_ref[0] = psum

    @pl.when(i != 0)
    def _():
        sum_ref[0] = sum_ref[0] + psum


def _pool(fg2):
    return pl.pallas_call(
        _pool_body,
        grid=(BS, N // BRF),
        compiler_params=pltpu.CompilerParams(
            dimension_semantics=("parallel", "arbitrary")),
        in_specs=[pl.BlockSpec((1, K, BRF, COUT), lambda b, i: (b, 0, i, 0))],
        out_specs=pl.BlockSpec((1, 1, COUT), lambda b, i: (b, 0, 0)),
        out_shape=jax.ShapeDtypeStruct((BS, 1, COUT), jnp.float32),
    )(fg2)


def _finish_body(sum_ref, f_ref, fste_ref, w2_ref, out_ref):
    gmean = sum_ref[0] * (1.0 / N)
    feat = f_ref[0]
    w1 = w2_ref[:COUT, :]
    wg = w2_ref[COUT:, :]
    fuse = (lax.dot_general(feat, w1, (((1,), (0,)), ((), ())),
                            preferred_element_type=jnp.float32)
            + lax.dot_general(gmean, wg, (((1,), (0,)), ((), ())),
                              preferred_element_type=jnp.float32))
    out_ref[0] = fuse + feat + fste_ref[0]


def _finish(fg2, feature, fste, W_conv2):
    sums = _pool(fg2)
    return pl.pallas_call(
        _finish_body,
        grid=(BS,),
        compiler_params=pltpu.CompilerParams(
            dimension_semantics=("parallel",)),
        in_specs=[
            pl.BlockSpec((1, 1, COUT), lambda b: (b, 0, 0)),
            pl.BlockSpec((1, N, COUT), lambda b: (b, 0, 0)),
            pl.BlockSpec((1, N, COUT), lambda b: (b, 0, 0)),
            pl.BlockSpec((2 * COUT, COUT), lambda b: (0, 0)),
        ],
        out_specs=pl.BlockSpec((1, N, COUT), lambda b: (b, 0, 0)),
        out_shape=jax.ShapeDtypeStruct((BS, N, COUT), jnp.float32),
    )(sums, feature, fste, W_conv2)


def kernel(vertices, feature_map, weights, bias, directions, W_ste, W_conv2,
           neighbor_num):
    bs, n, _ = vertices.shape
    vpad = jnp.pad(vertices, ((0, 0), (0, 0), (0, VPAD - 3)))
    vpadT = jnp.swapaxes(vpad, 1, 2)
    featT = jnp.swapaxes(feature_map, 1, 2)
    dirpad = jnp.pad(directions, ((0, VPAD - 3), (0, 0)))
    bias2 = bias.reshape(1, -1)

    idx, f_center, f_support, f_ste = _knn_feat_linear(
        feature_map, featT, weights, bias2, W_ste)
    idx2 = _knn_vert(vpad, vpadT)

    idx_t = jnp.swapaxes(idx, 1, 2).reshape(bs * n * K)
    vg, fsg = _sc_gather_pair(vpad.reshape(bs * n, VPAD),
                              f_support.reshape(bs * n, COUT), idx_t)
    feature = _combine(vg.reshape(bs, K, n, VPAD),
                       fsg.reshape(bs, K, n, COUT),
                       vpad, f_center, dirpad)

    idx2_t = jnp.swapaxes(idx2, 1, 2).reshape(bs * n * K)
    fg2 = _sc_gather_one(feature.reshape(bs * n, COUT), idx2_t)
    out = _finish(fg2.reshape(bs, K, n, COUT), feature, f_ste, W_conv2)

    delta = jnp.asarray(neighbor_num - K, out.dtype)
    return out + delta


# speedup vs baseline: 1.5150x; 1.5150x over previous
import functools

import jax
import jax.numpy as jnp
from jax import lax
from jax.experimental import pallas as pl
from jax.experimental.pallas import tpu as pltpu
from jax.experimental.pallas import tpu_sc as plsc

BS, N, CIN, COUT = 4, 4096, 64, 64
K = 16
BR = 512
BR2 = 1024
VPAD = 8
INF = float("inf")


def _topk_body(c, with_linear, fa_ref, frT_ref, *rest):
    if with_linear:
        fr_ref, w_ref, b_ref, ws_ref, idx_ref, fc_ref, fs_ref, fste_ref = rest
    else:
        (idx_ref,) = rest
    b = pl.program_id(0)
    i = pl.program_id(1)
    fa = fa_ref[0]
    frT = frT_ref[0]
    inner = lax.dot_general(fa, frT, (((1,), (0,)), ((), ())),
                            preferred_element_type=jnp.float32)
    qa = jnp.sum(fa * fa, axis=1, keepdims=True)
    qr = jnp.sum(frT * frT, axis=0, keepdims=True)
    d = inner * -2.0 + qa + qr
    lanev = lax.broadcasted_iota(jnp.int32, (N, BR), 1)
    rowv = lax.broadcasted_iota(jnp.int32, (N, BR), 0)
    d = jnp.where(rowv == lanev + i * BR, INF, d)
    sub16 = lax.broadcasted_iota(jnp.int32, (K, BR), 0)
    acc0 = jnp.zeros((K, BR), jnp.int32)

    G, GW = N // 128, 128
    sub32 = lax.broadcasted_iota(jnp.int32, (G, BR), 0)
    rowg = lax.broadcasted_iota(jnp.int32, (GW, BR), 0)
    gm_l, gc_l = [], []
    for g in range(G):
        sub = d[g * GW:(g + 1) * GW, :]
        m = jnp.min(sub, axis=0, keepdims=True)
        cidx = jnp.min(jnp.where(sub == m, rowg + g * GW, N), axis=0,
                       keepdims=True)
        gm_l.append(m)
        gc_l.append(cidx)
    gmin0 = jnp.concatenate(gm_l, axis=0)
    gcol0 = jnp.concatenate(gc_l, axis=0)

    def step(t, carry):
        gmin, gcol, acc = carry
        mv = jnp.min(gmin, axis=0, keepdims=True)
        am = jnp.min(jnp.where(gmin == mv, gcol, N), axis=0, keepdims=True)
        acc = jnp.where(sub16 == t, am, acc)
        gsel = lax.shift_right_logical(am, 7)
        level = [d[g * GW:(g + 1) * GW, :] for g in range(G)]
        nbit = 0
        while len(level) > 1:
            bit = lax.shift_right_logical(gsel, nbit) & 1
            level = [jnp.where(bit == 1, level[2 * i + 1], level[2 * i])
                     for i in range(len(level) // 2)]
            nbit += 1
        sel = level[0]
        rowsel = rowg + lax.shift_left(gsel, 7)
        ok = (sel > mv) | ((sel == mv) & (rowsel > am))
        selm = jnp.where(ok, sel, INF)
        nm = jnp.min(selm, axis=0, keepdims=True)
        nc = jnp.min(jnp.where(selm == nm, rowsel, N), axis=0, keepdims=True)
        gmin = jnp.where(sub32 == gsel, nm, gmin)
        gcol = jnp.where(sub32 == gsel, nc, gcol)
        return (gmin, gcol, acc)

    _, _, acc = lax.fori_loop(0, K, step, (gmin0, gcol0, acc0))
    idx_ref[0] = acc + b * N

    if with_linear:
        fr = fr_ref[0]
        fm2 = lax.dot_general(fr, w_ref[...], (((1,), (0,)), ((), ())),
                              preferred_element_type=jnp.float32) + b_ref[0][None, :]
        fc_ref[0] = fm2[:, :COUT]
        fs_ref[0] = fm2[:, COUT:]
        fste_ref[0] = lax.dot_general(fr, ws_ref[...], (((1,), (0,)), ((), ())),
                                      preferred_element_type=jnp.float32)


def _knn_feat_linear(feature_map, featT, weights, bias2, W_ste):
    grid = (BS, N // BR)
    body = functools.partial(_topk_body, CIN, True)
    return pl.pallas_call(
        body,
        grid=grid,
        compiler_params=pltpu.CompilerParams(
            dimension_semantics=("parallel", "parallel")),
        in_specs=[
            pl.BlockSpec((1, N, CIN), lambda b, i: (b, 0, 0)),
            pl.BlockSpec((1, CIN, BR), lambda b, i: (b, 0, i)),
            pl.BlockSpec((1, BR, CIN), lambda b, i: (b, i, 0)),
            pl.BlockSpec((CIN, 2 * COUT), lambda b, i: (0, 0)),
            pl.BlockSpec((1, 2 * COUT), lambda b, i: (0, 0)),
            pl.BlockSpec((CIN, COUT), lambda b, i: (0, 0)),
        ],
        out_specs=[
            pl.BlockSpec((1, K, BR), lambda b, i: (b, 0, i)),
            pl.BlockSpec((1, BR, COUT), lambda b, i: (b, i, 0)),
            pl.BlockSpec((1, BR, COUT), lambda b, i: (b, i, 0)),
            pl.BlockSpec((1, BR, COUT), lambda b, i: (b, i, 0)),
        ],
        out_shape=[
            jax.ShapeDtypeStruct((BS, K, N), jnp.int32),
            jax.ShapeDtypeStruct((BS, N, COUT), jnp.float32),
            jax.ShapeDtypeStruct((BS, N, COUT), jnp.float32),
            jax.ShapeDtypeStruct((BS, N, COUT), jnp.float32),
        ],
    )(feature_map, featT, feature_map, weights, bias2, W_ste)


def _knn_vert(vpad, vpadT):
    grid = (BS, N // BR)
    body = functools.partial(_topk_body, VPAD, False)
    return pl.pallas_call(
        body,
        grid=grid,
        compiler_params=pltpu.CompilerParams(
            dimension_semantics=("parallel", "parallel")),
        in_specs=[
            pl.BlockSpec((1, N, VPAD), lambda b, i: (b, 0, 0)),
            pl.BlockSpec((1, VPAD, BR), lambda b, i: (b, 0, i)),
        ],
        out_specs=pl.BlockSpec((1, K, BR), lambda b, i: (b, 0, i)),
        out_shape=jax.ShapeDtypeStruct((BS, K, N), jnp.int32),
    )(vpad, vpadT)


def _sc_gather_pair(table_v, table_s, idx):
    B = idx.shape[0]
    info = plsc.get_sparse_core_info()
    NC, NS = info.num_cores, info.num_subcores
    NW = NC * NS
    bpw = B // NW
    CH = 512
    mesh = plsc.VectorSubcoreMesh(core_axis_name="c", subcore_axis_name="s")

    @functools.partial(
        pl.kernel, mesh=mesh,
        compiler_params=pltpu.CompilerParams(use_tc_tiling_on_sc=False),
        out_type=(jax.ShapeDtypeStruct((B, VPAD), jnp.float32),
                  jax.ShapeDtypeStruct((B, COUT), jnp.float32)),
        scratch_types=[
            pltpu.VMEM((CH,), jnp.int32),
            pltpu.VMEM((CH, VPAD), jnp.float32),
            pltpu.VMEM((CH, COUT), jnp.float32),
            pltpu.SemaphoreType.DMA,
        ],
    )
    def k(tv_hbm, ts_hbm, idx_hbm, ov_hbm, os_hbm, idx_v, rv, rs, sem):
        wid = lax.axis_index("s") * NC + lax.axis_index("c")
        base = wid * bpw

        def body(c, carry):
            off = base + c * CH
            pltpu.sync_copy(idx_hbm.at[pl.ds(off, CH)], idx_v)
            pltpu.async_copy(tv_hbm.at[idx_v], rv, sem).wait()
            pltpu.async_copy(ts_hbm.at[idx_v], rs, sem).wait()
            pltpu.sync_copy(rv, ov_hbm.at[pl.ds(off, CH)])
            pltpu.sync_copy(rs, os_hbm.at[pl.ds(off, CH)])
            return carry

        lax.fori_loop(0, bpw // CH, body, 0)

    return k(table_v, table_s, idx)


def _sc_gather_one(table, idx):
    B = idx.shape[0]
    info = plsc.get_sparse_core_info()
    NC, NS = info.num_cores, info.num_subcores
    NW = NC * NS
    bpw = B // NW
    CH = 512
    mesh = plsc.VectorSubcoreMesh(core_axis_name="c", subcore_axis_name="s")

    @functools.partial(
        pl.kernel, mesh=mesh,
        compiler_params=pltpu.CompilerParams(use_tc_tiling_on_sc=False),
        out_type=jax.ShapeDtypeStruct((B, COUT), jnp.float32),
        scratch_types=[
            pltpu.VMEM((CH,), jnp.int32),
            pltpu.VMEM((CH, COUT), jnp.float32),
            pltpu.SemaphoreType.DMA,
        ],
    )
    def k(t_hbm, idx_hbm, o_hbm, idx_v, rows, sem):
        wid = lax.axis_index("s") * NC + lax.axis_index("c")
        base = wid * bpw

        def body(c, carry):
            off = base + c * CH
            pltpu.sync_copy(idx_hbm.at[pl.ds(off, CH)], idx_v)
            pltpu.async_copy(t_hbm.at[idx_v], rows, sem).wait()
            pltpu.sync_copy(rows, o_hbm.at[pl.ds(off, CH)])
            return carry

        lax.fori_loop(0, bpw // CH, body, 0)

    return k(table, idx)


def _combine_body(vg_ref, fsg_ref, vr_ref, fc_ref, dir_ref, out_ref):
    v = vr_ref[0]
    dirs = dir_ref[...]
    dn = jnp.sqrt(jnp.sum(dirs * dirs, axis=0, keepdims=True))
    sd = dirs / jnp.maximum(dn, 1e-12)
    acc = None
    for t in range(K):
        dv = vg_ref[0, t] - v
        nrm = jnp.sqrt(jnp.sum(dv * dv, axis=1, keepdims=True))
        rf = dv / jnp.maximum(nrm, 1e-12)
        theta = jnp.maximum(
            lax.dot_general(rf, sd, (((1,), (0,)), ((), ())),
                            preferred_element_type=jnp.float32), 0.0)
        a = theta * fsg_ref[0, t]
        acc = a if acc is None else jnp.maximum(acc, a)
    out_ref[0] = fc_ref[0] + acc


def _combine(vg4, fsg4, vpad, f_center, dirpad):
    grid = (BS, N // BR2)
    return pl.pallas_call(
        _combine_body,
        grid=grid,
        compiler_params=pltpu.CompilerParams(
            dimension_semantics=("parallel", "parallel")),
        in_specs=[
            pl.BlockSpec((1, K, BR2, VPAD), lambda b, i: (b, 0, i, 0)),
            pl.BlockSpec((1, K, BR2, COUT), lambda b, i: (b, 0, i, 0)),
            pl.BlockSpec((1, BR2, VPAD), lambda b, i: (b, i, 0)),
            pl.BlockSpec((1, BR2, COUT), lambda b, i: (b, i, 0)),
            pl.BlockSpec((VPAD, COUT), lambda b, i: (0, 0)),
        ],
        out_specs=pl.BlockSpec((1, BR2, COUT), lambda b, i: (b, i, 0)),
        out_shape=jax.ShapeDtypeStruct((BS, N, COUT), jnp.float32),
    )(vg4, fsg4, vpad, f_center, dirpad)


BRF = 1024


def _pool_body(fg2_ref, sum_ref):
    i = pl.program_id(1)
    fg = fg2_ref[0, 0]
    for t in range(1, K):
        fg = jnp.maximum(fg, fg2_ref[0, t])
    psum = jnp.sum(fg, axis=0, keepdims=True)

    @pl.when(i == 0)
    def _():
        sum_ref[0] = psum

    @pl.when(i != 0)
    def _():
        sum_ref[0] = sum_ref[0] + psum


def _pool(fg2):
    return pl.pallas_call(
        _pool_body,
        grid=(BS, N // BRF),
        compiler_params=pltpu.CompilerParams(
            dimension_semantics=("parallel", "arbitrary")),
        in_specs=[pl.BlockSpec((1, K, BRF, COUT), lambda b, i: (b, 0, i, 0))],
        out_specs=pl.BlockSpec((1, 1, COUT), lambda b, i: (b, 0, 0)),
        out_shape=jax.ShapeDtypeStruct((BS, 1, COUT), jnp.float32),
    )(fg2)


def _finish_body(sum_ref, f_ref, fste_ref, w2_ref, out_ref):
    gmean = sum_ref[0] * (1.0 / N)
    feat = f_ref[0]
    w1 = w2_ref[:COUT, :]
    wg = w2_ref[COUT:, :]
    fuse = (lax.dot_general(feat, w1, (((1,), (0,)), ((), ())),
                            preferred_element_type=jnp.float32)
            + lax.dot_general(gmean, wg, (((1,), (0,)), ((), ())),
                              preferred_element_type=jnp.float32))
    out_ref[0] = fuse + feat + fste_ref[0]


def _finish(fg2, feature, fste, W_conv2):
    sums = _pool(fg2)
    return pl.pallas_call(
        _finish_body,
        grid=(BS,),
        compiler_params=pltpu.CompilerParams(
            dimension_semantics=("parallel",)),
        in_specs=[
            pl.BlockSpec((1, 1, COUT), lambda b: (b, 0, 0)),
            pl.BlockSpec((1, N, COUT), lambda b: (b, 0, 0)),
            pl.BlockSpec((1, N, COUT), lambda b: (b, 0, 0)),
            pl.BlockSpec((2 * COUT, COUT), lambda b: (0, 0)),
        ],
        out_specs=pl.BlockSpec((1, N, COUT), lambda b: (b, 0, 0)),
        out_shape=jax.ShapeDtypeStruct((BS, N, COUT), jnp.float32),
    )(sums, feature, fste, W_conv2)


def kernel(vertices, feature_map, weights, bias, directions, W_ste, W_conv2,
           neighbor_num):
    bs, n, _ = vertices.shape
    vpad = jnp.pad(vertices, ((0, 0), (0, 0), (0, VPAD - 3)))
    vpadT = jnp.swapaxes(vpad, 1, 2)
    featT = jnp.swapaxes(feature_map, 1, 2)
    dirpad = jnp.pad(directions, ((0, VPAD - 3), (0, 0)))
    bias2 = bias.reshape(1, -1)

    idx, f_center, f_support, f_ste = _knn_feat_linear(
        feature_map, featT, weights, bias2, W_ste)
    idx2 = _knn_vert(vpad, vpadT)

    idx_t = idx.reshape(bs * n * K)
    vg, fsg = _sc_gather_pair(vpad.reshape(bs * n, VPAD),
                              f_support.reshape(bs * n, COUT), idx_t)
    feature = _combine(vg.reshape(bs, K, n, VPAD),
                       fsg.reshape(bs, K, n, COUT),
                       vpad, f_center, dirpad)

    idx2_t = idx2.reshape(bs * n * K)
    fg2 = _sc_gather_one(feature.reshape(bs * n, COUT), idx2_t)
    out = _finish(fg2.reshape(bs, K, n, COUT), feature, f_ste, W_conv2)

    delta = jnp.asarray(neighbor_num - K, out.dtype)
    return out + delta


# speedup vs baseline: 1.5369x; 1.0144x over previous
import functools

import jax
import jax.numpy as jnp
from jax import lax
from jax.experimental import pallas as pl
from jax.experimental.pallas import tpu as pltpu
from jax.experimental.pallas import tpu_sc as plsc

BS, N, CIN, COUT = 4, 4096, 64, 64
K = 16
BR = 512
BR2 = 1024
VPAD = 8
INF = float("inf")


def _topk_body(c, with_linear, fa_ref, frT_ref, *rest):
    if with_linear:
        fr_ref, w_ref, b_ref, ws_ref, idx_ref, fc_ref, fs_ref, fste_ref = rest
    else:
        (idx_ref,) = rest
    b = pl.program_id(0)
    i = pl.program_id(1)
    fa = fa_ref[0]
    frT = frT_ref[0]
    inner = lax.dot_general(fa, frT * -2.0, (((1,), (0,)), ((), ())),
                            preferred_element_type=jnp.float32)
    qa = jnp.sum(fa * fa, axis=1, keepdims=True)
    qr = jnp.sum(frT * frT, axis=0, keepdims=True)
    d = inner + qa + qr
    lanerow = lax.broadcasted_iota(jnp.int32, (1, BR), 1)
    sub16 = lax.broadcasted_iota(jnp.int32, (K, BR), 0)
    acc0 = jnp.zeros((K, BR), jnp.int32)

    G, GW = N // 128, 128
    sub32 = lax.broadcasted_iota(jnp.int32, (G, BR), 0)
    rowg = lax.broadcasted_iota(jnp.int32, (GW, BR), 0)
    gm_l, gc_l = [], []
    for g in range(G):
        sub = d[g * GW:(g + 1) * GW, :]
        sub = jnp.where(rowg + g * GW == lanerow + i * BR, INF, sub)
        m = jnp.min(sub, axis=0, keepdims=True)
        cidx = jnp.min(jnp.where(sub == m, rowg + g * GW, N), axis=0,
                       keepdims=True)
        gm_l.append(m)
        gc_l.append(cidx)
    gmin0 = jnp.concatenate(gm_l, axis=0)
    gcol0 = jnp.concatenate(gc_l, axis=0)

    def step(t, carry):
        gmin, gcol, acc = carry
        mv = jnp.min(gmin, axis=0, keepdims=True)
        am = jnp.min(jnp.where(gmin == mv, gcol, N), axis=0, keepdims=True)
        acc = jnp.where(sub16 == t, am, acc)
        gsel = lax.shift_right_logical(am, 7)
        level = [d[g * GW:(g + 1) * GW, :] for g in range(G)]
        nbit = 0
        while len(level) > 1:
            bit = lax.shift_right_logical(gsel, nbit) & 1
            level = [jnp.where(bit == 1, level[2 * i + 1], level[2 * i])
                     for i in range(len(level) // 2)]
            nbit += 1
        sel = level[0]
        rowsel = rowg + lax.shift_left(gsel, 7)
        ok = (sel > mv) | ((sel == mv) & (rowsel > am))
        selm = jnp.where(ok, sel, INF)
        nm = jnp.min(selm, axis=0, keepdims=True)
        nc = jnp.min(jnp.where(selm == nm, rowsel, N), axis=0, keepdims=True)
        gmin = jnp.where(sub32 == gsel, nm, gmin)
        gcol = jnp.where(sub32 == gsel, nc, gcol)
        return (gmin, gcol, acc)

    _, _, acc = lax.fori_loop(0, K, step, (gmin0, gcol0, acc0))
    idx_ref[0] = acc + b * N

    if with_linear:
        fr = fr_ref[0]
        fm2 = lax.dot_general(fr, w_ref[...], (((1,), (0,)), ((), ())),
                              preferred_element_type=jnp.float32) + b_ref[0][None, :]
        fc_ref[0] = fm2[:, :COUT]
        fs_ref[0] = fm2[:, COUT:]
        fste_ref[0] = lax.dot_general(fr, ws_ref[...], (((1,), (0,)), ((), ())),
                                      preferred_element_type=jnp.float32)


def _knn_feat_linear(feature_map, featT, weights, bias2, W_ste):
    grid = (BS, N // BR)
    body = functools.partial(_topk_body, CIN, True)
    return pl.pallas_call(
        body,
        grid=grid,
        compiler_params=pltpu.CompilerParams(
            dimension_semantics=("parallel", "parallel")),
        in_specs=[
            pl.BlockSpec((1, N, CIN), lambda b, i: (b, 0, 0)),
            pl.BlockSpec((1, CIN, BR), lambda b, i: (b, 0, i)),
            pl.BlockSpec((1, BR, CIN), lambda b, i: (b, i, 0)),
            pl.BlockSpec((CIN, 2 * COUT), lambda b, i: (0, 0)),
            pl.BlockSpec((1, 2 * COUT), lambda b, i: (0, 0)),
            pl.BlockSpec((CIN, COUT), lambda b, i: (0, 0)),
        ],
        out_specs=[
            pl.BlockSpec((1, K, BR), lambda b, i: (b, 0, i)),
            pl.BlockSpec((1, BR, COUT), lambda b, i: (b, i, 0)),
            pl.BlockSpec((1, BR, COUT), lambda b, i: (b, i, 0)),
            pl.BlockSpec((1, BR, COUT), lambda b, i: (b, i, 0)),
        ],
        out_shape=[
            jax.ShapeDtypeStruct((BS, K, N), jnp.int32),
            jax.ShapeDtypeStruct((BS, N, COUT), jnp.float32),
            jax.ShapeDtypeStruct((BS, N, COUT), jnp.float32),
            jax.ShapeDtypeStruct((BS, N, COUT), jnp.float32),
        ],
    )(feature_map, featT, feature_map, weights, bias2, W_ste)


def _knn_vert(vpad, vpadT):
    grid = (BS, N // BR)
    body = functools.partial(_topk_body, VPAD, False)
    return pl.pallas_call(
        body,
        grid=grid,
        compiler_params=pltpu.CompilerParams(
            dimension_semantics=("parallel", "parallel")),
        in_specs=[
            pl.BlockSpec((1, N, VPAD), lambda b, i: (b, 0, 0)),
            pl.BlockSpec((1, VPAD, BR), lambda b, i: (b, 0, i)),
        ],
        out_specs=pl.BlockSpec((1, K, BR), lambda b, i: (b, 0, i)),
        out_shape=jax.ShapeDtypeStruct((BS, K, N), jnp.int32),
    )(vpad, vpadT)


def _sc_gather_pair(table_v, table_s, idx):
    B = idx.shape[0]
    info = plsc.get_sparse_core_info()
    NC, NS = info.num_cores, info.num_subcores
    NW = NC * NS
    bpw = B // NW
    CH = 512
    mesh = plsc.VectorSubcoreMesh(core_axis_name="c", subcore_axis_name="s")

    @functools.partial(
        pl.kernel, mesh=mesh,
        compiler_params=pltpu.CompilerParams(use_tc_tiling_on_sc=False),
        out_type=(jax.ShapeDtypeStruct((B, VPAD), jnp.float32),
                  jax.ShapeDtypeStruct((B, COUT), jnp.float32)),
        scratch_types=[
            pltpu.VMEM((CH,), jnp.int32),
            pltpu.VMEM((CH, VPAD), jnp.float32),
            pltpu.VMEM((CH, COUT), jnp.float32),
            pltpu.SemaphoreType.DMA,
        ],
    )
    def k(tv_hbm, ts_hbm, idx_hbm, ov_hbm, os_hbm, idx_v, rv, rs, sem):
        wid = lax.axis_index("s") * NC + lax.axis_index("c")
        base = wid * bpw

        def body(c, carry):
            off = base + c * CH
            pltpu.sync_copy(idx_hbm.at[pl.ds(off, CH)], idx_v)
            pltpu.async_copy(tv_hbm.at[idx_v], rv, sem).wait()
            pltpu.async_copy(ts_hbm.at[idx_v], rs, sem).wait()
            pltpu.sync_copy(rv, ov_hbm.at[pl.ds(off, CH)])
            pltpu.sync_copy(rs, os_hbm.at[pl.ds(off, CH)])
            return carry

        lax.fori_loop(0, bpw // CH, body, 0)

    return k(table_v, table_s, idx)


def _sc_gather_one(table, idx):
    B = idx.shape[0]
    info = plsc.get_sparse_core_info()
    NC, NS = info.num_cores, info.num_subcores
    NW = NC * NS
    bpw = B // NW
    CH = 512
    mesh = plsc.VectorSubcoreMesh(core_axis_name="c", subcore_axis_name="s")

    @functools.partial(
        pl.kernel, mesh=mesh,
        compiler_params=pltpu.CompilerParams(use_tc_tiling_on_sc=False),
        out_type=jax.ShapeDtypeStruct((B, COUT), jnp.float32),
        scratch_types=[
            pltpu.VMEM((CH,), jnp.int32),
            pltpu.VMEM((CH, COUT), jnp.float32),
            pltpu.SemaphoreType.DMA,
        ],
    )
    def k(t_hbm, idx_hbm, o_hbm, idx_v, rows, sem):
        wid = lax.axis_index("s") * NC + lax.axis_index("c")
        base = wid * bpw

        def body(c, carry):
            off = base + c * CH
            pltpu.sync_copy(idx_hbm.at[pl.ds(off, CH)], idx_v)
            pltpu.async_copy(t_hbm.at[idx_v], rows, sem).wait()
            pltpu.sync_copy(rows, o_hbm.at[pl.ds(off, CH)])
            return carry

        lax.fori_loop(0, bpw // CH, body, 0)

    return k(table, idx)


def _combine_body(vg_ref, fsg_ref, vr_ref, fc_ref, dir_ref, out_ref):
    v = vr_ref[0]
    dirs = dir_ref[...]
    dn = jnp.sqrt(jnp.sum(dirs * dirs, axis=0, keepdims=True))
    sd = dirs / jnp.maximum(dn, 1e-12)
    acc = None
    for t in range(K):
        dv = vg_ref[0, t] - v
        nrm = jnp.sqrt(jnp.sum(dv * dv, axis=1, keepdims=True))
        rf = dv / jnp.maximum(nrm, 1e-12)
        theta = jnp.maximum(
            lax.dot_general(rf, sd, (((1,), (0,)), ((), ())),
                            preferred_element_type=jnp.float32), 0.0)
        a = theta * fsg_ref[0, t]
        acc = a if acc is None else jnp.maximum(acc, a)
    out_ref[0] = fc_ref[0] + acc


def _combine(vg4, fsg4, vpad, f_center, dirpad):
    grid = (BS, N // BR2)
    return pl.pallas_call(
        _combine_body,
        grid=grid,
        compiler_params=pltpu.CompilerParams(
            dimension_semantics=("parallel", "parallel")),
        in_specs=[
            pl.BlockSpec((1, K, BR2, VPAD), lambda b, i: (b, 0, i, 0)),
            pl.BlockSpec((1, K, BR2, COUT), lambda b, i: (b, 0, i, 0)),
            pl.BlockSpec((1, BR2, VPAD), lambda b, i: (b, i, 0)),
            pl.BlockSpec((1, BR2, COUT), lambda b, i: (b, i, 0)),
            pl.BlockSpec((VPAD, COUT), lambda b, i: (0, 0)),
        ],
        out_specs=pl.BlockSpec((1, BR2, COUT), lambda b, i: (b, i, 0)),
        out_shape=jax.ShapeDtypeStruct((BS, N, COUT), jnp.float32),
    )(vg4, fsg4, vpad, f_center, dirpad)


BRF = 1024


def _pool_body(fg2_ref, sum_ref):
    i = pl.program_id(1)
    fg = fg2_ref[0, 0]
    for t in range(1, K):
        fg = jnp.maximum(fg, fg2_ref[0, t])
    psum = jnp.sum(fg, axis=0, keepdims=True)

    @pl.when(i == 0)
    def _():
        sum_ref[0] = psum

    @pl.when(i != 0)
    def _():
        sum_ref[0] = sum_ref[0] + psum


def _pool(fg2):
    return pl.pallas_call(
        _pool_body,
        grid=(BS, N // BRF),
        compiler_params=pltpu.CompilerParams(
            dimension_semantics=("parallel", "arbitrary")),
        in_specs=[pl.BlockSpec((1, K, BRF, COUT), lambda b, i: (b, 0, i, 0))],
        out_specs=pl.BlockSpec((1, 1, COUT), lambda b, i: (b, 0, 0)),
        out_shape=jax.ShapeDtypeStruct((BS, 1, COUT), jnp.float32),
    )(fg2)


def _finish_body(sum_ref, f_ref, fste_ref, w2_ref, out_ref):
    gmean = sum_ref[0] * (1.0 / N)
    feat = f_ref[0]
    w1 = w2_ref[:COUT, :]
    wg = w2_ref[COUT:, :]
    fuse = (lax.dot_general(feat, w1, (((1,), (0,)), ((), ())),
                            preferred_element_type=jnp.float32)
            + lax.dot_general(gmean, wg, (((1,), (0,)), ((), ())),
                              preferred_element_type=jnp.float32))
    out_ref[0] = fuse + feat + fste_ref[0]


def _finish(fg2, feature, fste, W_conv2):
    sums = _pool(fg2)
    return pl.pallas_call(
        _finish_body,
        grid=(BS,),
        compiler_params=pltpu.CompilerParams(
            dimension_semantics=("parallel",)),
        in_specs=[
            pl.BlockSpec((1, 1, COUT), lambda b: (b, 0, 0)),
            pl.BlockSpec((1, N, COUT), lambda b: (b, 0, 0)),
            pl.BlockSpec((1, N, COUT), lambda b: (b, 0, 0)),
            pl.BlockSpec((2 * COUT, COUT), lambda b: (0, 0)),
        ],
        out_specs=pl.BlockSpec((1, N, COUT), lambda b: (b, 0, 0)),
        out_shape=jax.ShapeDtypeStruct((BS, N, COUT), jnp.float32),
    )(sums, feature, fste, W_conv2)


def kernel(vertices, feature_map, weights, bias, directions, W_ste, W_conv2,
           neighbor_num):
    bs, n, _ = vertices.shape
    vpad = jnp.pad(vertices, ((0, 0), (0, 0), (0, VPAD - 3)))
    vpadT = jnp.swapaxes(vpad, 1, 2)
    featT = jnp.swapaxes(feature_map, 1, 2)
    dirpad = jnp.pad(directions, ((0, VPAD - 3), (0, 0)))
    bias2 = bias.reshape(1, -1)

    idx, f_center, f_support, f_ste = _knn_feat_linear(
        feature_map, featT, weights, bias2, W_ste)
    idx2 = _knn_vert(vpad, vpadT)

    idx_t = idx.reshape(bs * n * K)
    vg, fsg = _sc_gather_pair(vpad.reshape(bs * n, VPAD),
                              f_support.reshape(bs * n, COUT), idx_t)
    feature = _combine(vg.reshape(bs, K, n, VPAD),
                       fsg.reshape(bs, K, n, COUT),
                       vpad, f_center, dirpad)

    idx2_t = idx2.reshape(bs * n * K)
    fg2 = _sc_gather_one(feature.reshape(bs * n, COUT), idx2_t)
    out = _finish(fg2.reshape(bs, K, n, COUT), feature, f_ste, W_conv2)

    delta = jnp.asarray(neighbor_num - K, out.dtype)
    return out + delta


# speedup vs baseline: 1.7771x; 1.1563x over previous
import functools

import jax
import jax.numpy as jnp
from jax import lax
from jax.experimental import pallas as pl
from jax.experimental.pallas import tpu as pltpu
from jax.experimental.pallas import tpu_sc as plsc

BS, N, CIN, COUT = 4, 4096, 64, 64
K = 16
BR = 512
BR2 = 1024
VPAD = 8
INF = float("inf")


def _topk_body(c, with_linear, fa_ref, frT_ref, *rest):
    if with_linear:
        (fr_ref, vp_ref, w_ref, b_ref, ws_ref, idx_ref, fc_ref, tab_ref,
         fste_ref) = rest
    else:
        (idx_ref,) = rest
    b = pl.program_id(0)
    i = pl.program_id(1)
    fa = fa_ref[0]
    frT = frT_ref[0]
    inner = lax.dot_general(fa, frT * -2.0, (((1,), (0,)), ((), ())),
                            preferred_element_type=jnp.float32)
    qa = jnp.sum(fa * fa, axis=1, keepdims=True)
    qr = jnp.sum(frT * frT, axis=0, keepdims=True)
    d = inner + qa + qr
    lanerow = lax.broadcasted_iota(jnp.int32, (1, BR), 1)
    sub16 = lax.broadcasted_iota(jnp.int32, (K, BR), 0)
    acc0 = jnp.zeros((K, BR), jnp.int32)

    G, GW = N // 128, 128
    sub32 = lax.broadcasted_iota(jnp.int32, (G, BR), 0)
    rowg = lax.broadcasted_iota(jnp.int32, (GW, BR), 0)
    gm_l, gc_l = [], []
    for g in range(G):
        sub = d[g * GW:(g + 1) * GW, :]
        sub = jnp.where(rowg + g * GW == lanerow + i * BR, INF, sub)
        m = jnp.min(sub, axis=0, keepdims=True)
        cidx = jnp.min(jnp.where(sub == m, rowg + g * GW, N), axis=0,
                       keepdims=True)
        gm_l.append(m)
        gc_l.append(cidx)
    gmin0 = jnp.concatenate(gm_l, axis=0)
    gcol0 = jnp.concatenate(gc_l, axis=0)

    def step(t, carry):
        gmin, gcol, acc = carry
        mv = jnp.min(gmin, axis=0, keepdims=True)
        am = jnp.min(jnp.where(gmin == mv, gcol, N), axis=0, keepdims=True)
        acc = jnp.where(sub16 == t, am, acc)
        gsel = lax.shift_right_logical(am, 7)
        level = [d[g * GW:(g + 1) * GW, :] for g in range(G)]
        nbit = 0
        while len(level) > 1:
            bit = lax.shift_right_logical(gsel, nbit) & 1
            level = [jnp.where(bit == 1, level[2 * i + 1], level[2 * i])
                     for i in range(len(level) // 2)]
            nbit += 1
        sel = level[0]
        rowsel = rowg + lax.shift_left(gsel, 7)
        ok = (sel > mv) | ((sel == mv) & (rowsel > am))
        selm = jnp.where(ok, sel, INF)
        nm = jnp.min(selm, axis=0, keepdims=True)
        nc = jnp.min(jnp.where(selm == nm, rowsel, N), axis=0, keepdims=True)
        gmin = jnp.where(sub32 == gsel, nm, gmin)
        gcol = jnp.where(sub32 == gsel, nc, gcol)
        return (gmin, gcol, acc)

    _, _, acc = lax.fori_loop(0, K, step, (gmin0, gcol0, acc0))
    idx_ref[0] = acc + b * N

    if with_linear:
        fr = fr_ref[0]
        vp = vp_ref[0]
        fm2 = lax.dot_general(fr, w_ref[...], (((1,), (0,)), ((), ())),
                              preferred_element_type=jnp.float32) + b_ref[0][None, :]
        fc_ref[0] = fm2[:, :COUT]
        zpad = jnp.zeros((BR, 128 - COUT - VPAD), jnp.float32)
        tab_ref[0] = jnp.concatenate([fm2[:, COUT:], vp, zpad], axis=1)
        fste_ref[0] = lax.dot_general(fr, ws_ref[...], (((1,), (0,)), ((), ())),
                                      preferred_element_type=jnp.float32)


def _knn_feat_linear(feature_map, featT, vpad, weights, bias2, W_ste):
    grid = (BS, N // BR)
    body = functools.partial(_topk_body, CIN, True)
    return pl.pallas_call(
        body,
        grid=grid,
        compiler_params=pltpu.CompilerParams(
            dimension_semantics=("parallel", "parallel")),
        in_specs=[
            pl.BlockSpec((1, N, CIN), lambda b, i: (b, 0, 0)),
            pl.BlockSpec((1, CIN, BR), lambda b, i: (b, 0, i)),
            pl.BlockSpec((1, BR, CIN), lambda b, i: (b, i, 0)),
            pl.BlockSpec((1, BR, VPAD), lambda b, i: (b, i, 0)),
            pl.BlockSpec((CIN, 2 * COUT), lambda b, i: (0, 0)),
            pl.BlockSpec((1, 2 * COUT), lambda b, i: (0, 0)),
            pl.BlockSpec((CIN, COUT), lambda b, i: (0, 0)),
        ],
        out_specs=[
            pl.BlockSpec((1, K, BR), lambda b, i: (b, 0, i)),
            pl.BlockSpec((1, BR, COUT), lambda b, i: (b, i, 0)),
            pl.BlockSpec((1, BR, 128), lambda b, i: (b, i, 0)),
            pl.BlockSpec((1, BR, COUT), lambda b, i: (b, i, 0)),
        ],
        out_shape=[
            jax.ShapeDtypeStruct((BS, K, N), jnp.int32),
            jax.ShapeDtypeStruct((BS, N, COUT), jnp.float32),
            jax.ShapeDtypeStruct((BS, N, 128), jnp.float32),
            jax.ShapeDtypeStruct((BS, N, COUT), jnp.float32),
        ],
    )(feature_map, featT, feature_map, vpad, weights, bias2, W_ste)


def _knn_vert(vpad, vpadT):
    grid = (BS, N // BR)
    body = functools.partial(_topk_body, VPAD, False)
    return pl.pallas_call(
        body,
        grid=grid,
        compiler_params=pltpu.CompilerParams(
            dimension_semantics=("parallel", "parallel")),
        in_specs=[
            pl.BlockSpec((1, N, VPAD), lambda b, i: (b, 0, 0)),
            pl.BlockSpec((1, VPAD, BR), lambda b, i: (b, 0, i)),
        ],
        out_specs=pl.BlockSpec((1, K, BR), lambda b, i: (b, 0, i)),
        out_shape=jax.ShapeDtypeStruct((BS, K, N), jnp.int32),
    )(vpad, vpadT)


def _sc_gather128(table, idx):
    B = idx.shape[0]
    info = plsc.get_sparse_core_info()
    NC, NS = info.num_cores, info.num_subcores
    NW = NC * NS
    bpw = B // NW
    CH = 512
    mesh = plsc.VectorSubcoreMesh(core_axis_name="c", subcore_axis_name="s")

    @functools.partial(
        pl.kernel, mesh=mesh,
        out_type=jax.ShapeDtypeStruct((B, 128), jnp.float32),
        scratch_types=[
            pltpu.VMEM((CH,), jnp.int32),
            pltpu.VMEM((CH, 128), jnp.float32),
            pltpu.SemaphoreType.DMA,
        ],
    )
    def k(t_hbm, idx_hbm, o_hbm, idx_v, rows, sem):
        wid = lax.axis_index("s") * NC + lax.axis_index("c")
        base = wid * bpw

        def body(c, carry):
            off = base + c * CH
            pltpu.sync_copy(idx_hbm.at[pl.ds(off, CH)], idx_v)
            pltpu.async_copy(t_hbm.at[idx_v], rows, sem).wait()
            pltpu.sync_copy(rows, o_hbm.at[pl.ds(off, CH)])
            return carry

        lax.fori_loop(0, bpw // CH, body, 0)

    return k(table, idx)


def _combine_body(g_ref, vr_ref, fc_ref, dir_ref, out_ref):
    v = vr_ref[0]
    dirs = dir_ref[...]
    dn = jnp.sqrt(jnp.sum(dirs * dirs, axis=0, keepdims=True))
    sd = dirs / jnp.maximum(dn, 1e-12)
    acc = None
    for t in range(K):
        blk = g_ref[0, t]
        dv = blk[:, COUT:COUT + VPAD] - v
        nrm = jnp.sqrt(jnp.sum(dv * dv, axis=1, keepdims=True))
        rf = dv / jnp.maximum(nrm, 1e-12)
        theta = jnp.maximum(
            lax.dot_general(rf, sd, (((1,), (0,)), ((), ())),
                            preferred_element_type=jnp.float32), 0.0)
        a = theta * blk[:, :COUT]
        acc = a if acc is None else jnp.maximum(acc, a)
    feat = fc_ref[0] + acc
    zpad = jnp.zeros((BR2, 128 - COUT), jnp.float32)
    out_ref[0] = jnp.concatenate([feat, zpad], axis=1)


def _combine(g4, vpad, f_center, dirpad):
    grid = (BS, N // BR2)
    return pl.pallas_call(
        _combine_body,
        grid=grid,
        compiler_params=pltpu.CompilerParams(
            dimension_semantics=("parallel", "parallel")),
        in_specs=[
            pl.BlockSpec((1, K, BR2, 128), lambda b, i: (b, 0, i, 0)),
            pl.BlockSpec((1, BR2, VPAD), lambda b, i: (b, i, 0)),
            pl.BlockSpec((1, BR2, COUT), lambda b, i: (b, i, 0)),
            pl.BlockSpec((VPAD, COUT), lambda b, i: (0, 0)),
        ],
        out_specs=pl.BlockSpec((1, BR2, 128), lambda b, i: (b, i, 0)),
        out_shape=jax.ShapeDtypeStruct((BS, N, 128), jnp.float32),
    )(g4, vpad, f_center, dirpad)


BRF = 1024


def _pool_body(fg2_ref, sum_ref):
    i = pl.program_id(1)
    fg = fg2_ref[0, 0][:, :COUT]
    for t in range(1, K):
        fg = jnp.maximum(fg, fg2_ref[0, t][:, :COUT])
    psum = jnp.sum(fg, axis=0, keepdims=True)

    @pl.when(i == 0)
    def _():
        sum_ref[0] = psum

    @pl.when(i != 0)
    def _():
        sum_ref[0] = sum_ref[0] + psum


def _pool(fg2):
    return pl.pallas_call(
        _pool_body,
        grid=(BS, N // BRF),
        compiler_params=pltpu.CompilerParams(
            dimension_semantics=("parallel", "arbitrary")),
        in_specs=[pl.BlockSpec((1, K, BRF, 128), lambda b, i: (b, 0, i, 0))],
        out_specs=pl.BlockSpec((1, 1, COUT), lambda b, i: (b, 0, 0)),
        out_shape=jax.ShapeDtypeStruct((BS, 1, COUT), jnp.float32),
    )(fg2)


def _finish_body(sum_ref, f_ref, fste_ref, w2_ref, out_ref):
    gmean = sum_ref[0] * (1.0 / N)
    feat = f_ref[0][:, :COUT]
    w1 = w2_ref[:COUT, :]
    wg = w2_ref[COUT:, :]
    fuse = (lax.dot_general(feat, w1, (((1,), (0,)), ((), ())),
                            preferred_element_type=jnp.float32)
            + lax.dot_general(gmean, wg, (((1,), (0,)), ((), ())),
                              preferred_element_type=jnp.float32))
    out_ref[0] = fuse + feat + fste_ref[0]


def _finish(fg2, feature, fste, W_conv2):
    sums = _pool(fg2)
    return pl.pallas_call(
        _finish_body,
        grid=(BS,),
        compiler_params=pltpu.CompilerParams(
            dimension_semantics=("parallel",)),
        in_specs=[
            pl.BlockSpec((1, 1, COUT), lambda b: (b, 0, 0)),
            pl.BlockSpec((1, N, 128), lambda b: (b, 0, 0)),
            pl.BlockSpec((1, N, COUT), lambda b: (b, 0, 0)),
            pl.BlockSpec((2 * COUT, COUT), lambda b: (0, 0)),
        ],
        out_specs=pl.BlockSpec((1, N, COUT), lambda b: (b, 0, 0)),
        out_shape=jax.ShapeDtypeStruct((BS, N, COUT), jnp.float32),
    )(sums, feature, fste, W_conv2)


def kernel(vertices, feature_map, weights, bias, directions, W_ste, W_conv2,
           neighbor_num):
    bs, n, _ = vertices.shape
    vpad = jnp.pad(vertices, ((0, 0), (0, 0), (0, VPAD - 3)))
    vpadT = jnp.swapaxes(vpad, 1, 2)
    featT = jnp.swapaxes(feature_map, 1, 2)
    dirpad = jnp.pad(directions, ((0, VPAD - 3), (0, 0)))
    bias2 = bias.reshape(1, -1)

    idx, f_center, table, f_ste = _knn_feat_linear(
        feature_map, featT, vpad, weights, bias2, W_ste)
    idx2 = _knn_vert(vpad, vpadT)

    idx_t = idx.reshape(bs * n * K)
    g = _sc_gather128(table.reshape(bs * n, 128), idx_t)
    feature = _combine(g.reshape(bs, K, n, 128), vpad, f_center, dirpad)

    idx2_t = idx2.reshape(bs * n * K)
    fg2 = _sc_gather128(feature.reshape(bs * n, 128), idx2_t)
    out = _finish(fg2.reshape(bs, K, n, 128), feature, f_ste, W_conv2)

    delta = jnp.asarray(neighbor_num - K, out.dtype)
    return out + delta
